# Initial kernel scaffold; baseline (speedup 1.0000x reference)
import jax
import jax.numpy as jnp
from jax.experimental import pallas as pl


def kernel(x, params, edge_index, raw_edge_feat):
    raise NotImplementedError("write your pallas kernel here")


# speedup vs baseline: 3.9893x; 3.9893x over previous
import functools

import jax
import jax.numpy as jnp
from jax import lax
from jax.experimental import pallas as pl
from jax.experimental.pallas import tpu as pltpu
from jax.experimental.pallas import tpu_sc as plsc

_N = 8192
_E = 131072
_K = 16
_LATENT = 128
_EDGE_DIM = 16
_HIDDEN = 256

_NC = 2
_NS = 16
_NW = _NC * _NS
_CH = 128
_GROWS = _E // _NW
_NCH = _GROWS // _CH


def _mesh():
    return plsc.VectorSubcoreMesh(core_axis_name="c", subcore_axis_name="s")


def _sc_gather(node, dst2d):
    nburst = 4
    wrows = nburst * _CH

    @functools.partial(
        pl.kernel,
        out_type=jax.ShapeDtypeStruct((_E, _LATENT), jnp.float32),
        mesh=_mesh(),
        compiler_params=pltpu.CompilerParams(use_tc_tiling_on_sc=False),
        scratch_types=[
            pltpu.VMEM((_NCH, _CH), jnp.int32),
            pltpu.VMEM((wrows, _LATENT), jnp.float32),
            pltpu.SemaphoreType.DMA,
        ],
    )
    def k(node_hbm, idx_hbm, out_hbm, idx_v, rows_v, sem):
        cid = lax.axis_index("c")
        sid = lax.axis_index("s")
        wid = cid * _NS + sid
        pltpu.sync_copy(
            idx_hbm.at[pl.ds(pl.multiple_of(wid * _NCH, _NCH), _NCH)], idx_v
        )

        def outer(g, carry):
            copies = [
                pltpu.async_copy(
                    node_hbm.at[idx_v.at[g * nburst + b]],
                    rows_v.at[pl.ds(b * _CH, _CH)],
                    sem,
                )
                for b in range(nburst)
            ]
            for c in copies:
                c.wait()
            pltpu.sync_copy(
                rows_v,
                out_hbm.at[
                    pl.ds(pl.multiple_of(wid * _GROWS + g * wrows, wrows), wrows)
                ],
            )
            return carry

        lax.fori_loop(0, _NCH // nburst, outer, 0)

    return k(node, dst2d)


def _sc_scatter(edge, dst2d, zeros):
    erows = _E // _NW
    nch = erows // _CH
    zrows = _N // _NS

    @functools.partial(
        pl.kernel,
        out_type=jax.ShapeDtypeStruct((_NC, _N, _EDGE_DIM), jnp.float32),
        mesh=_mesh(),
        compiler_params=pltpu.CompilerParams(use_tc_tiling_on_sc=False),
        scratch_types=[
            pltpu.VMEM((erows, _EDGE_DIM), jnp.float32),
            pltpu.VMEM((nch, _CH), jnp.int32),
            pltpu.VMEM_SHARED((_N, _EDGE_DIM), jnp.float32),
            pltpu.SemaphoreType.DMA,
        ],
    )
    def k(edge_hbm, idx_hbm, zeros_hbm, out_hbm, ebuf, idx_v, acc, sem):
        cid = lax.axis_index("c")
        sid = lax.axis_index("s")
        erow0 = pl.multiple_of(cid * (_E // _NC) + sid * erows, erows)
        zrow0 = pl.multiple_of(sid * zrows, zrows)
        pltpu.sync_copy(
            zeros_hbm.at[pl.ds(zrow0, zrows)],
            acc.at[pl.ds(zrow0, zrows)],
        )
        pltpu.sync_copy(edge_hbm.at[pl.ds(erow0, erows)], ebuf)
        pltpu.sync_copy(
            idx_hbm.at[pl.ds(pl.multiple_of(erow0 // _CH, nch), nch)], idx_v
        )
        plsc.subcore_barrier()

        def body(j, carry):
            pltpu.sync_copy(
                ebuf.at[pl.ds(j * _CH, _CH)], acc.at[idx_v.at[j]], add=True
            )
            return carry

        lax.fori_loop(0, nch, body, 0)
        plsc.subcore_barrier()
        pltpu.sync_copy(
            acc.at[pl.ds(zrow0, zrows)],
            out_hbm.at[cid, pl.ds(zrow0, zrows)],
        )

    return k(edge, dst2d, zeros)


def _full(shape):
    return pl.BlockSpec(shape, lambda i: (0,) * len(shape))


def _tc_mlp2(xn, w1, b1, w2, b2, bn):
    n, din = xn.shape
    dh = w1.shape[1]
    dout = w2.shape[1]

    def body(x_ref, w1r, b1r, w2r, b2r, o_ref):
        h = jnp.dot(x_ref[...], w1r[...], preferred_element_type=jnp.float32)
        h = h + b1r[...]
        h = h * jax.nn.sigmoid(h)
        o_ref[...] = (
            jnp.dot(h, w2r[...], preferred_element_type=jnp.float32) + b2r[...]
        )

    return pl.pallas_call(
        body,
        grid=(n // bn,),
        in_specs=[
            pl.BlockSpec((bn, din), lambda i: (i, 0)),
            _full((din, dh)),
            _full((1, dh)),
            _full((dh, dout)),
            _full((1, dout)),
        ],
        out_specs=pl.BlockSpec((bn, dout), lambda i: (i, 0)),
        out_shape=jax.ShapeDtypeStruct((n, dout), jnp.float32),
    )(xn, w1, b1.reshape(1, -1), w2, b2.reshape(1, -1))


def _ln(v, g, b):
    m = jnp.mean(v, axis=-1, keepdims=True)
    var = jnp.mean((v - m) ** 2, axis=-1, keepdims=True)
    return (v - m) * lax.rsqrt(var + 1e-5) * g + b


_BE = 2048
_BNODE = _BE // _K


def _tc_edge_layer(node, df, edge, w1a, w1b, w1c, b1, w2, b2, g, beta):
    def body(n_ref, df_ref, e_ref, w1ar, w1br, w1cr, b1r, w2r, b2r, gr, br,
             o_ref):
        a = jnp.dot(n_ref[...], w1ar[...], preferred_element_type=jnp.float32)
        rep = jnp.broadcast_to(
            a[:, None, :], (_BNODE, _K, _HIDDEN)
        ).reshape(_BE, _HIDDEN)
        pre = rep + jnp.dot(
            df_ref[...], w1br[...], preferred_element_type=jnp.float32
        )
        pre = pre + jnp.dot(
            e_ref[...], w1cr[...], preferred_element_type=jnp.float32
        )
        pre = pre + b1r[...]
        h = pre * jax.nn.sigmoid(pre)
        e2 = e_ref[...] + (
            jnp.dot(h, w2r[...], preferred_element_type=jnp.float32) + b2r[...]
        )
        o_ref[...] = _ln(e2, gr[...], br[...])

    return pl.pallas_call(
        body,
        grid=(_E // _BE,),
        in_specs=[
            pl.BlockSpec((_BNODE, _LATENT), lambda i: (i, 0)),
            pl.BlockSpec((_BE, _LATENT), lambda i: (i, 0)),
            pl.BlockSpec((_BE, _EDGE_DIM), lambda i: (i, 0)),
            _full((_LATENT, _HIDDEN)),
            _full((_LATENT, _HIDDEN)),
            _full((_EDGE_DIM, _HIDDEN)),
            _full((1, _HIDDEN)),
            _full((_HIDDEN, _EDGE_DIM)),
            _full((1, _EDGE_DIM)),
            _full((1, _EDGE_DIM)),
            _full((1, _EDGE_DIM)),
        ],
        out_specs=pl.BlockSpec((_BE, _EDGE_DIM), lambda i: (i, 0)),
        out_shape=jax.ShapeDtypeStruct((_E, _EDGE_DIM), jnp.float32),
    )(node, df, edge, w1a, w1b, w1c, b1.reshape(1, -1), w2,
      b2.reshape(1, -1), g.reshape(1, -1), beta.reshape(1, -1))


_BN = 1024


def _tc_node_layer(node, agg0, agg1, w1a, w1b, b1, w2, b2, g, beta):
    def body(n_ref, a0, a1, w1ar, w1br, b1r, w2r, b2r, gr, br, o_ref):
        agg = a0[...] + a1[...]
        pre = jnp.dot(n_ref[...], w1ar[...], preferred_element_type=jnp.float32)
        pre = pre + jnp.dot(
            agg, w1br[...], preferred_element_type=jnp.float32
        )
        pre = pre + b1r[...]
        h = pre * jax.nn.sigmoid(pre)
        n2 = n_ref[...] + (
            jnp.dot(h, w2r[...], preferred_element_type=jnp.float32) + b2r[...]
        )
        o_ref[...] = _ln(n2, gr[...], br[...])

    return pl.pallas_call(
        body,
        grid=(_N // _BN,),
        in_specs=[
            pl.BlockSpec((_BN, _LATENT), lambda i: (i, 0)),
            pl.BlockSpec((_BN, _EDGE_DIM), lambda i: (i, 0)),
            pl.BlockSpec((_BN, _EDGE_DIM), lambda i: (i, 0)),
            _full((_LATENT, _HIDDEN)),
            _full((_EDGE_DIM, _HIDDEN)),
            _full((1, _HIDDEN)),
            _full((_HIDDEN, _LATENT)),
            _full((1, _LATENT)),
            _full((1, _LATENT)),
            _full((1, _LATENT)),
        ],
        out_specs=pl.BlockSpec((_BN, _LATENT), lambda i: (i, 0)),
        out_shape=jax.ShapeDtypeStruct((_N, _LATENT), jnp.float32),
    )(node, agg0, agg1, w1a, w1b, b1.reshape(1, -1), w2, b2.reshape(1, -1),
      g.reshape(1, -1), beta.reshape(1, -1))


def kernel(x, params, edge_index, raw_edge_feat):
    B, C, H, W = x.shape
    xn = jnp.transpose(x.reshape(C, _N))
    dst = edge_index[1].astype(jnp.int32)
    dst2d = dst.reshape(_E // _CH, _CH)
    zeros = jnp.zeros((_N, _EDGE_DIM), jnp.float32)

    pe = params["node_enc"]
    node = _tc_mlp2(xn, pe["W1"], pe["b1"], pe["W2"], pe["b2"], 1024)
    pe = params["edge_enc"]
    edge = _tc_mlp2(raw_edge_feat, pe["W1"], pe["b1"], pe["W2"], pe["b2"], 8192)

    stacked = jax.tree.map(lambda *xs: jnp.stack(xs), *params["layers"])

    def layer(carry, lp):
        node, edge = carry
        df = _sc_gather(node, dst2d)
        edge = _tc_edge_layer(
            node, df, edge,
            lp["eW1"][:_LATENT], lp["eW1"][_LATENT:2 * _LATENT],
            lp["eW1"][2 * _LATENT:], lp["eb1"], lp["eW2"], lp["eb2"],
            lp["eg"], lp["ebeta"],
        )
        parts = _sc_scatter(edge, dst2d, zeros)
        node = _tc_node_layer(
            node, parts[0], parts[1],
            lp["nW1"][:_LATENT], lp["nW1"][_LATENT:], lp["nb1"],
            lp["nW2"], lp["nb2"], lp["ng"], lp["nbeta"],
        )
        return (node, edge), None

    (node, edge), _ = lax.scan(layer, (node, edge), stacked)

    pd = params["dec"]
    out = _tc_mlp2(node, pd["W1"], pd["b1"], pd["W2"], pd["b2"], 1024)
    return jnp.transpose(out)[None].reshape(B, out.shape[-1], H, W)
